# Initial kernel scaffold; baseline (speedup 1.0000x reference)
import jax
import jax.numpy as jnp
from jax.experimental import pallas as pl


def kernel(params, node_type_id_mat, node_type_id_atom, partition_mat, partition_atom, node_ids_atom, edge_m2a, edge_a2m):
    raise NotImplementedError("write your pallas kernel here")


# speedup vs baseline: 1.9623x; 1.9623x over previous
import functools

import jax
import jax.numpy as jnp
from jax import lax
from jax.experimental import pallas as pl
from jax.experimental.pallas import tpu as pltpu
from jax.experimental.pallas import tpu_sc as plsc

N = 50000
H = 128
CH = 16
NCH = 8
E = 300000
NS = 16
B = 128

BPT = 147
E_PAD = NS * BPT * B
EROWS = NS * BPT
DUMP = N
ACC_ROWS = N + 8
RPT = N // NS

BPT_C = 74
E_PAD_C = 32 * BPT_C * B
EROWS_C = 32 * BPT_C

def _segsum_body(x0, x1, x2, x3, x4, x5, x6, x7, srcr, dstr,
                 o0, o1, o2, o3, o4, o5, o6, o7,
                 src_v, dst_v, rows_v, zb_v, acc, sem):
    c = lax.axis_index("c")
    s = lax.axis_index("s")
    xs = (x0, x1, x2, x3, x4, x5, x6, x7)
    outs = (o0, o1, o2, o3, o4, o5, o6, o7)

    zero16 = jnp.zeros((16,), jnp.float32)

    def zfill(i, carry):
        zb_v[i] = zero16
        return carry

    lax.fori_loop(0, 625, zfill, 0)

    pltpu.sync_copy(srcr.at[s], src_v)
    pltpu.sync_copy(dstr.at[s], dst_v)

    r0 = s * RPT
    for cc in range(2):
        @pl.when(c == cc)
        def _(cc=cc):
            for k in range(4):
                g = cc * 4 + k
                xg = xs[g]
                og = outs[g]
                for q in range(5):
                    pltpu.sync_copy(zb_v, acc.at[pl.ds(r0 + q * 625, 625)])
                plsc.subcore_barrier()

                def batch(j, carry):
                    pltpu.async_copy(xg.at[src_v.at[j]], rows_v, sem).wait()
                    pltpu.sync_copy(rows_v, acc.at[dst_v.at[j]], add=True)
                    return carry

                lax.fori_loop(0, BPT, batch, 0)
                plsc.subcore_barrier()
                pltpu.sync_copy(acc.at[pl.ds(r0, RPT)], og.at[s])


@functools.cache
def _sc_mesh():
    return plsc.VectorSubcoreMesh(core_axis_name="c", subcore_axis_name="s")


@functools.cache
def _segsum_kernel():
    return pl.kernel(
        _segsum_body,
        out_type=[jax.ShapeDtypeStruct((NS, RPT, CH), jnp.float32)] * 8,
        mesh=_sc_mesh(),
        compiler_params=pltpu.CompilerParams(use_tc_tiling_on_sc=False),
        scratch_types=[
            pltpu.VMEM((BPT, B), jnp.int32),
            pltpu.VMEM((BPT, B), jnp.int32),
            pltpu.VMEM((B, CH), jnp.float32),
            pltpu.VMEM((625, CH), jnp.float32),
            pltpu.VMEM_SHARED((ACC_ROWS, CH), jnp.float32),
            pltpu.SemaphoreType.DMA,
        ],
    )


def _count_body(dstr, out, dst_v, ones_v, zb_v, acc):
    c = lax.axis_index("c")
    s = lax.axis_index("s")
    w = c * NS + s
    one16 = jnp.full((16,), 1.0, jnp.float32)
    zero16 = jnp.zeros((16,), jnp.float32)

    def ofill(i, carry):
        ones_v[i] = one16
        return carry

    lax.fori_loop(0, B, ofill, 0)

    def zfill(i, carry):
        zb_v[i] = zero16
        return carry

    lax.fori_loop(0, 625, zfill, 0)

    pltpu.sync_copy(dstr.at[w], dst_v)
    r0 = s * RPT
    for q in range(5):
        pltpu.sync_copy(zb_v, acc.at[pl.ds(r0 + q * 625, 625)])
    plsc.subcore_barrier()

    def batch(j, carry):
        pltpu.sync_copy(ones_v, acc.at[dst_v.at[j]], add=True)
        return carry

    lax.fori_loop(0, BPT_C, batch, 0)
    plsc.subcore_barrier()
    pltpu.sync_copy(acc.at[pl.ds(r0, RPT)], out.at[w])


@functools.cache
def _count_kernel():
    return pl.kernel(
        _count_body,
        out_type=jax.ShapeDtypeStruct((2 * NS, RPT, 16), jnp.float32),
        mesh=_sc_mesh(),
        compiler_params=pltpu.CompilerParams(use_tc_tiling_on_sc=False),
        scratch_types=[
            pltpu.VMEM((BPT_C, B), jnp.int32),
            pltpu.VMEM((B, 16), jnp.float32),
            pltpu.VMEM((625, 16), jnp.float32),
            pltpu.VMEM_SHARED((ACC_ROWS, 16), jnp.float32),
        ],
    )


R = 2000
GRID = N // R


def _init_mat_body(part_ref, table_ref, o_ref):
    p = part_ref[...]
    oh = (p == lax.broadcasted_iota(jnp.int32, (R, 4), 1)).astype(jnp.float32)
    o_ref[...] = jnp.dot(oh, table_ref[...], preferred_element_type=jnp.float32)


def _init_atom_body(part_ref, table_ref, wn_ref, o_ref):
    p = part_ref[...]
    oh = (p == lax.broadcasted_iota(jnp.int32, (R, 4), 1)).astype(jnp.float32)
    o_ref[...] = (jnp.dot(oh, table_ref[...], preferred_element_type=jnp.float32)
                  + wn_ref[...])


def _sage_body(s_ref, cnt_ref, x_ref, wl_ref, wr_ref, b_ref, o_ref):
    cnt = cnt_ref[0] + cnt_ref[1]
    inv = 1.0 / jnp.maximum(cnt[:, 0:1], 1.0)
    mean = s_ref[...] * inv
    o_ref[...] = jnp.maximum(
        jnp.dot(mean, wl_ref[...], preferred_element_type=jnp.float32)
        + jnp.dot(x_ref[...], wr_ref[...], preferred_element_type=jnp.float32)
        + b_ref[...], 0.0)


def _ffw_body(x_ref, w0_ref, w1_ref, w2_ref, b0_ref, b1_ref, b2_ref, o_ref):
    h = x_ref[...]
    h = jnp.maximum(jnp.dot(h, w0_ref[...], preferred_element_type=jnp.float32)
                    + b0_ref[...], 0.0)
    h = jnp.maximum(jnp.dot(h, w1_ref[...], preferred_element_type=jnp.float32)
                    + b1_ref[...], 0.0)
    o_ref[...] = jnp.maximum(
        jnp.dot(h, w2_ref[...], preferred_element_type=jnp.float32)
        + b2_ref[...], 0.0)


def _rows_spec(width):
    return pl.BlockSpec((R, width), lambda i: (i, 0))


def _full_spec(shape):
    nd = len(shape)
    return pl.BlockSpec(shape, lambda i: (0,) * nd)


_init_mat = pl.pallas_call(
    _init_mat_body,
    grid=(GRID,),
    in_specs=[_rows_spec(1), _full_spec((4, H))],
    out_specs=_rows_spec(H),
    out_shape=jax.ShapeDtypeStruct((N, H), jnp.float32),
)

_init_atom = pl.pallas_call(
    _init_atom_body,
    grid=(GRID,),
    in_specs=[_rows_spec(1), _full_spec((4, H)), _rows_spec(H)],
    out_specs=_rows_spec(H),
    out_shape=jax.ShapeDtypeStruct((N, H), jnp.float32),
)

_sage_tc = pl.pallas_call(
    _sage_body,
    grid=(GRID,),
    in_specs=[
        _rows_spec(H),
        pl.BlockSpec((2, R, 16), lambda i: (0, i, 0)),
        _rows_spec(H),
        _full_spec((H, H)),
        _full_spec((H, H)),
        _full_spec((1, H)),
    ],
    out_specs=_rows_spec(H),
    out_shape=jax.ShapeDtypeStruct((N, H), jnp.float32),
)

_ffw_tc = pl.pallas_call(
    _ffw_body,
    grid=(GRID,),
    in_specs=[_rows_spec(H)] + [_full_spec((H, H))] * 3 + [_full_spec((1, H))] * 3,
    out_specs=_rows_spec(H),
    out_shape=jax.ShapeDtypeStruct((N, H), jnp.float32),
)


def _pad_edges(src, dst, total, rows):
    pe = total - E
    src_p = jnp.concatenate([src.astype(jnp.int32), jnp.zeros((pe,), jnp.int32)])
    dst_p = jnp.concatenate([dst.astype(jnp.int32),
                             jnp.full((pe,), DUMP, jnp.int32)])
    return src_p.reshape(NS, rows // NS, B), dst_p.reshape(NS, rows // NS, B)


def _pad_dst(dst):
    pe = E_PAD_C - E
    d = jnp.concatenate([dst.astype(jnp.int32), jnp.full((pe,), DUMP, jnp.int32)])
    return d.reshape(32, BPT_C, B)


def _chunks(x):
    return tuple(x[:, k * CH:(k + 1) * CH] for k in range(NCH))


def _agg(x, src_r, dst_r):
    outs = _segsum_kernel()(*_chunks(x), src_r, dst_r)
    return jnp.concatenate([o.reshape(N, CH) for o in outs], axis=1)


def kernel(params, node_type_id_mat, node_type_id_atom, partition_mat,
           partition_atom, node_ids_atom, edge_m2a, edge_a2m):
    p = params
    table_m = p["W_type"][0] + p["W_part"][:, 0, :]
    table_a = p["W_type"][1] + p["W_part"][:, 1, :]

    part_m = partition_mat.astype(jnp.int32).reshape(N, 1)
    part_a = partition_atom.astype(jnp.int32).reshape(N, 1)

    z_mat = _init_mat(part_m, table_m)
    z_atom = _init_atom(part_a, table_a, p["W_node_atom"])

    src_a2m, dst_a2m = _pad_edges(edge_a2m[0], edge_a2m[1], E_PAD, EROWS)
    src_m2a, dst_m2a = _pad_edges(edge_m2a[0], edge_m2a[1], E_PAD, EROWS)

    cnt = _count_kernel()
    cnt_mat = cnt(_pad_dst(edge_a2m[1])).reshape(2, N, 16)
    cnt_atom = cnt(_pad_dst(edge_m2a[1])).reshape(2, N, 16)

    x_mat, x_atom = z_mat, z_atom
    for i in range(2):
        s_mat = _agg(x_atom, src_a2m, dst_a2m)
        s_atom = _agg(x_mat, src_m2a, dst_m2a)
        nm = _sage_tc(s_mat, cnt_mat, x_mat,
                      p["Wl_a2m_%d" % i].T, p["Wr_a2m_%d" % i].T,
                      p["bl_a2m_%d" % i].reshape(1, H))
        na = _sage_tc(s_atom, cnt_atom, x_atom,
                      p["Wl_m2a_%d" % i].T, p["Wr_m2a_%d" % i].T,
                      p["bl_m2a_%d" % i].reshape(1, H))
        x_mat, x_atom = nm, na

    x_mat = _ffw_tc(x_mat, p["Wf_mat_0"].T, p["Wf_mat_1"].T, p["Wf_mat_2"].T,
                    p["bf_mat_0"].reshape(1, H), p["bf_mat_1"].reshape(1, H),
                    p["bf_mat_2"].reshape(1, H))
    x_atom = _ffw_tc(x_atom, p["Wf_atom_0"].T, p["Wf_atom_1"].T,
                     p["Wf_atom_2"].T, p["bf_atom_0"].reshape(1, H),
                     p["bf_atom_1"].reshape(1, H), p["bf_atom_2"].reshape(1, H))
    return (x_mat, x_atom)


# speedup vs baseline: 2.4753x; 1.2614x over previous
import functools

import jax
import jax.numpy as jnp
from jax import lax
from jax.experimental import pallas as pl
from jax.experimental.pallas import tpu as pltpu
from jax.experimental.pallas import tpu_sc as plsc

N = 50000
H = 128
CH = 16
NCH = 8
E = 300000
NS = 16
B = 128

BPT = 147
E_PAD = NS * BPT * B
EROWS = NS * BPT
DUMP = N
ACC_ROWS = N + 8
RPT = N // NS

BPT_C = 74
E_PAD_C = 32 * BPT_C * B
EROWS_C = 32 * BPT_C

def _segsum_body(x0, x1, x2, x3, x4, x5, x6, x7, srcr, dstr,
                 o0, o1, o2, o3, o4, o5, o6, o7,
                 src_v, dst_v, rows0_v, rows1_v, zb_v, acc, sem0, sem1):
    c = lax.axis_index("c")
    s = lax.axis_index("s")
    xs = (x0, x1, x2, x3, x4, x5, x6, x7)
    outs = (o0, o1, o2, o3, o4, o5, o6, o7)

    zero16 = jnp.zeros((16,), jnp.float32)

    def zfill(i, carry):
        zb_v[i] = zero16
        return carry

    lax.fori_loop(0, 625, zfill, 0)

    pltpu.sync_copy(srcr.at[s], src_v)
    pltpu.sync_copy(dstr.at[s], dst_v)

    r0 = s * RPT
    for cc in range(2):
        @pl.when(c == cc)
        def _(cc=cc):
            for k in range(4):
                g = cc * 4 + k
                xg = xs[g]
                og = outs[g]
                for q in range(5):
                    pltpu.sync_copy(zb_v, acc.at[pl.ds(r0 + q * 625, 625)])
                plsc.subcore_barrier()

                pltpu.async_copy(xg.at[src_v.at[0]], rows0_v, sem0)

                def batch(j, carry):
                    nxt = j + 1

                    @pl.when(j % 2 == 0)
                    def _even():
                        @pl.when(nxt < BPT)
                        def _():
                            pltpu.async_copy(xg.at[src_v.at[nxt]],
                                             rows1_v, sem1)
                        pltpu.make_async_copy(xg.at[src_v.at[j]],
                                              rows0_v, sem0).wait()
                        pltpu.sync_copy(rows0_v, acc.at[dst_v.at[j]],
                                        add=True)

                    @pl.when(j % 2 == 1)
                    def _odd():
                        @pl.when(nxt < BPT)
                        def _():
                            pltpu.async_copy(xg.at[src_v.at[nxt]],
                                             rows0_v, sem0)
                        pltpu.make_async_copy(xg.at[src_v.at[j]],
                                              rows1_v, sem1).wait()
                        pltpu.sync_copy(rows1_v, acc.at[dst_v.at[j]],
                                        add=True)

                    return carry

                lax.fori_loop(0, BPT, batch, 0)
                plsc.subcore_barrier()
                pltpu.sync_copy(acc.at[pl.ds(r0, RPT)], og.at[s])


@functools.cache
def _sc_mesh():
    return plsc.VectorSubcoreMesh(core_axis_name="c", subcore_axis_name="s")


@functools.cache
def _segsum_kernel():
    return pl.kernel(
        _segsum_body,
        out_type=[jax.ShapeDtypeStruct((NS, RPT, CH), jnp.float32)] * 8,
        mesh=_sc_mesh(),
        compiler_params=pltpu.CompilerParams(use_tc_tiling_on_sc=False),
        scratch_types=[
            pltpu.VMEM((BPT, B), jnp.int32),
            pltpu.VMEM((BPT, B), jnp.int32),
            pltpu.VMEM((B, CH), jnp.float32),
            pltpu.VMEM((B, CH), jnp.float32),
            pltpu.VMEM((625, CH), jnp.float32),
            pltpu.VMEM_SHARED((ACC_ROWS, CH), jnp.float32),
            pltpu.SemaphoreType.DMA,
            pltpu.SemaphoreType.DMA,
        ],
    )


def _count_body(dstr, out, dst_v, ones_v, zb_v, acc):
    c = lax.axis_index("c")
    s = lax.axis_index("s")
    w = c * NS + s
    one16 = jnp.full((16,), 1.0, jnp.float32)
    zero16 = jnp.zeros((16,), jnp.float32)

    def ofill(i, carry):
        ones_v[i] = one16
        return carry

    lax.fori_loop(0, B, ofill, 0)

    def zfill(i, carry):
        zb_v[i] = zero16
        return carry

    lax.fori_loop(0, 625, zfill, 0)

    pltpu.sync_copy(dstr.at[w], dst_v)
    r0 = s * RPT
    for q in range(5):
        pltpu.sync_copy(zb_v, acc.at[pl.ds(r0 + q * 625, 625)])
    plsc.subcore_barrier()

    def batch(j, carry):
        pltpu.sync_copy(ones_v, acc.at[dst_v.at[j]], add=True)
        return carry

    lax.fori_loop(0, BPT_C, batch, 0)
    plsc.subcore_barrier()
    pltpu.sync_copy(acc.at[pl.ds(r0, RPT)], out.at[w])


@functools.cache
def _count_kernel():
    return pl.kernel(
        _count_body,
        out_type=jax.ShapeDtypeStruct((2 * NS, RPT, 16), jnp.float32),
        mesh=_sc_mesh(),
        compiler_params=pltpu.CompilerParams(use_tc_tiling_on_sc=False),
        scratch_types=[
            pltpu.VMEM((BPT_C, B), jnp.int32),
            pltpu.VMEM((B, 16), jnp.float32),
            pltpu.VMEM((625, 16), jnp.float32),
            pltpu.VMEM_SHARED((ACC_ROWS, 16), jnp.float32),
        ],
    )


R = 2000
GRID = N // R


def _init_mat_body(part_ref, table_ref, o_ref):
    p = part_ref[...]
    oh = (p == lax.broadcasted_iota(jnp.int32, (R, 4), 1)).astype(jnp.float32)
    o_ref[...] = jnp.dot(oh, table_ref[...], preferred_element_type=jnp.float32)


def _init_atom_body(part_ref, table_ref, wn_ref, o_ref):
    p = part_ref[...]
    oh = (p == lax.broadcasted_iota(jnp.int32, (R, 4), 1)).astype(jnp.float32)
    o_ref[...] = (jnp.dot(oh, table_ref[...], preferred_element_type=jnp.float32)
                  + wn_ref[...])


def _sage_body(s_ref, cnt_ref, x_ref, wl_ref, wr_ref, b_ref, o_ref):
    cnt = cnt_ref[0] + cnt_ref[1]
    inv = 1.0 / jnp.maximum(cnt[:, 0:1], 1.0)
    mean = s_ref[...] * inv
    o_ref[...] = jnp.maximum(
        jnp.dot(mean, wl_ref[...], preferred_element_type=jnp.float32)
        + jnp.dot(x_ref[...], wr_ref[...], preferred_element_type=jnp.float32)
        + b_ref[...], 0.0)


def _ffw_body(x_ref, w0_ref, w1_ref, w2_ref, b0_ref, b1_ref, b2_ref, o_ref):
    h = x_ref[...]
    h = jnp.maximum(jnp.dot(h, w0_ref[...], preferred_element_type=jnp.float32)
                    + b0_ref[...], 0.0)
    h = jnp.maximum(jnp.dot(h, w1_ref[...], preferred_element_type=jnp.float32)
                    + b1_ref[...], 0.0)
    o_ref[...] = jnp.maximum(
        jnp.dot(h, w2_ref[...], preferred_element_type=jnp.float32)
        + b2_ref[...], 0.0)


def _rows_spec(width):
    return pl.BlockSpec((R, width), lambda i: (i, 0))


def _full_spec(shape):
    nd = len(shape)
    return pl.BlockSpec(shape, lambda i: (0,) * nd)


_init_mat = pl.pallas_call(
    _init_mat_body,
    grid=(GRID,),
    in_specs=[_rows_spec(1), _full_spec((4, H))],
    out_specs=_rows_spec(H),
    out_shape=jax.ShapeDtypeStruct((N, H), jnp.float32),
)

_init_atom = pl.pallas_call(
    _init_atom_body,
    grid=(GRID,),
    in_specs=[_rows_spec(1), _full_spec((4, H)), _rows_spec(H)],
    out_specs=_rows_spec(H),
    out_shape=jax.ShapeDtypeStruct((N, H), jnp.float32),
)

_sage_tc = pl.pallas_call(
    _sage_body,
    grid=(GRID,),
    in_specs=[
        _rows_spec(H),
        pl.BlockSpec((2, R, 16), lambda i: (0, i, 0)),
        _rows_spec(H),
        _full_spec((H, H)),
        _full_spec((H, H)),
        _full_spec((1, H)),
    ],
    out_specs=_rows_spec(H),
    out_shape=jax.ShapeDtypeStruct((N, H), jnp.float32),
)

_ffw_tc = pl.pallas_call(
    _ffw_body,
    grid=(GRID,),
    in_specs=[_rows_spec(H)] + [_full_spec((H, H))] * 3 + [_full_spec((1, H))] * 3,
    out_specs=_rows_spec(H),
    out_shape=jax.ShapeDtypeStruct((N, H), jnp.float32),
)


def _pad_edges(src, dst, total, rows):
    pe = total - E
    src_p = jnp.concatenate([src.astype(jnp.int32), jnp.zeros((pe,), jnp.int32)])
    dst_p = jnp.concatenate([dst.astype(jnp.int32),
                             jnp.full((pe,), DUMP, jnp.int32)])
    return src_p.reshape(NS, rows // NS, B), dst_p.reshape(NS, rows // NS, B)


def _pad_dst(dst):
    pe = E_PAD_C - E
    d = jnp.concatenate([dst.astype(jnp.int32), jnp.full((pe,), DUMP, jnp.int32)])
    return d.reshape(32, BPT_C, B)


def _chunks(x):
    return tuple(x[:, k * CH:(k + 1) * CH] for k in range(NCH))


def _agg(x, src_r, dst_r):
    outs = _segsum_kernel()(*_chunks(x), src_r, dst_r)
    return jnp.concatenate([o.reshape(N, CH) for o in outs], axis=1)


def kernel(params, node_type_id_mat, node_type_id_atom, partition_mat,
           partition_atom, node_ids_atom, edge_m2a, edge_a2m):
    p = params
    table_m = p["W_type"][0] + p["W_part"][:, 0, :]
    table_a = p["W_type"][1] + p["W_part"][:, 1, :]

    part_m = partition_mat.astype(jnp.int32).reshape(N, 1)
    part_a = partition_atom.astype(jnp.int32).reshape(N, 1)

    z_mat = _init_mat(part_m, table_m)
    z_atom = _init_atom(part_a, table_a, p["W_node_atom"])

    src_a2m, dst_a2m = _pad_edges(edge_a2m[0], edge_a2m[1], E_PAD, EROWS)
    src_m2a, dst_m2a = _pad_edges(edge_m2a[0], edge_m2a[1], E_PAD, EROWS)

    cnt = _count_kernel()
    cnt_mat = cnt(_pad_dst(edge_a2m[1])).reshape(2, N, 16)
    cnt_atom = cnt(_pad_dst(edge_m2a[1])).reshape(2, N, 16)

    x_mat, x_atom = z_mat, z_atom
    for i in range(2):
        s_mat = _agg(x_atom, src_a2m, dst_a2m)
        s_atom = _agg(x_mat, src_m2a, dst_m2a)
        nm = _sage_tc(s_mat, cnt_mat, x_mat,
                      p["Wl_a2m_%d" % i].T, p["Wr_a2m_%d" % i].T,
                      p["bl_a2m_%d" % i].reshape(1, H))
        na = _sage_tc(s_atom, cnt_atom, x_atom,
                      p["Wl_m2a_%d" % i].T, p["Wr_m2a_%d" % i].T,
                      p["bl_m2a_%d" % i].reshape(1, H))
        x_mat, x_atom = nm, na

    x_mat = _ffw_tc(x_mat, p["Wf_mat_0"].T, p["Wf_mat_1"].T, p["Wf_mat_2"].T,
                    p["bf_mat_0"].reshape(1, H), p["bf_mat_1"].reshape(1, H),
                    p["bf_mat_2"].reshape(1, H))
    x_atom = _ffw_tc(x_atom, p["Wf_atom_0"].T, p["Wf_atom_1"].T,
                     p["Wf_atom_2"].T, p["bf_atom_0"].reshape(1, H),
                     p["bf_atom_1"].reshape(1, H), p["bf_atom_2"].reshape(1, H))
    return (x_mat, x_atom)
